# baseline (speedup 1.0000x reference)
import jax
import jax.numpy as jnp
from jax import lax
from jax.experimental import pallas as pl

_B = 16384
_D = 1000
_R = 1024


def _onehot_body(x_ref, o_ref):
    idx = x_ref[...]
    cols = lax.broadcasted_iota(jnp.int32, o_ref.shape, 1)
    o_ref[...] = (cols == idx).astype(jnp.float32)


def kernel(x):
    x = x.reshape(_B, 1).astype(jnp.int32)
    return pl.pallas_call(
        _onehot_body,
        grid=(_B // _R,),
        in_specs=[pl.BlockSpec((_R, 1), lambda i: (i, 0))],
        out_specs=pl.BlockSpec((_R, _D), lambda i: (i, 0)),
        out_shape=jax.ShapeDtypeStruct((_B, _D), jnp.float32),
    )(x)
